# baseline (speedup 1.0000x reference)
import functools

import jax
import jax.numpy as jnp
from jax import lax
from jax.experimental import pallas as pl
from jax.experimental.pallas import tpu as pltpu

KB = 4096
SUB = 1024
TOPK = 10
RUN = 16
BIG = 2**31 - 1


def _dist_subtile(q, q_sq, kb, ki, s, k_total):
    kb_s = kb[s * SUB:(s + 1) * SUB, :]
    dots = lax.dot_general(q, kb_s, (((1,), (1,)), ((), ())),
                           preferred_element_type=jnp.float32)
    ones = jnp.ones((1, 128), jnp.float32)
    k_sq = lax.dot_general(ones, kb_s * kb_s, (((1,), (1,)), ((), ())),
                           preferred_element_type=jnp.float32,
                           precision=lax.Precision.HIGHEST)
    base = ki * KB + s * SUB
    col1 = base + lax.broadcasted_iota(jnp.int32, (1, SUB), 1)
    k_sq = jnp.where(col1 < k_total, k_sq, jnp.inf)
    return (q_sq + k_sq) - 2.0 * dots, base


def _full_pops(dists, bases, rv, ri, qb):
    nv = jnp.full((qb, RUN), jnp.inf, jnp.float32)
    ni = jnp.zeros((qb, RUN), jnp.int32)
    lane16 = lax.broadcasted_iota(jnp.int32, (qb, RUN), 1)
    dists = list(dists)
    for t in range(TOPK):
        m = jnp.min(rv, axis=1)
        for d in dists:
            m = jnp.minimum(m, jnp.min(d, axis=1))
        mq = m[:, None]
        sel = jnp.min(jnp.where(rv == mq, ri, BIG), axis=1)
        for j, d in enumerate(dists):
            colj = bases[j] + lax.broadcasted_iota(jnp.int32, (qb, SUB), 1)
            sel = jnp.minimum(
                sel, jnp.min(jnp.where(d == mq, colj, BIG), axis=1))
        selq = sel[:, None]
        rv = jnp.where(ri == selq, jnp.inf, rv)
        for j, d in enumerate(dists):
            colj = bases[j] + lax.broadcasted_iota(jnp.int32, (qb, SUB), 1)
            dists[j] = jnp.where(colj == selq, jnp.inf, d)
        nv = jnp.where(lane16 == t, mq, nv)
        ni = jnp.where(lane16 == t, selq, ni)
    return nv, ni


def _body(q_ref, kb_ref, d_ref, i_ref, p_ref, runv_ref, runi_ref, *,
          kg, k_total, qb):
    ki = pl.program_id(1)

    q = q_ref[...]
    kb = kb_ref[...]
    q_sq = jnp.sum(q * q, axis=1, keepdims=True)

    inf16 = jnp.full((qb, RUN), jnp.inf, jnp.float32)
    rv = jnp.where(ki == 0, inf16, runv_ref[...])
    ri = jnp.where(ki == 0, jnp.zeros((qb, RUN), jnp.int32), runi_ref[...])

    lane = lax.broadcasted_iota(jnp.int32, (qb, 128), 1)
    infs = jnp.full((qb, 128), jnp.inf, jnp.float32)
    zi = jnp.zeros((qb, 128), jnp.int32)
    p0, p1, p2, p3 = infs, infs, infs, infs
    q0, q1, q2, q3 = zi, zi, zi, zi

    for s in range(KB // SUB):
        dist_s, base = _dist_subtile(q, q_sq, kb, ki, s, k_total)
        for g in range(SUB // 128):
            d = dist_s[:, g * 128:(g + 1) * 128]
            cid = (base + g * 128) + lane
            c1 = d < p0
            t1 = jnp.maximum(d, p0)
            ti1 = jnp.where(c1, q0, cid)
            p0 = jnp.minimum(d, p0)
            q0 = jnp.where(c1, cid, q0)
            c2 = t1 < p1
            t2 = jnp.maximum(t1, p1)
            ti2 = jnp.where(c2, q1, ti1)
            p1 = jnp.minimum(t1, p1)
            q1 = jnp.where(c2, ti1, q1)
            c3 = t2 < p2
            t3 = jnp.maximum(t2, p2)
            ti3 = jnp.where(c3, q2, ti2)
            p2 = jnp.minimum(t2, p2)
            q2 = jnp.where(c3, ti2, q2)
            c4 = t3 < p3
            p3 = jnp.minimum(t3, p3)
            q3 = jnp.where(c4, ti3, q3)

    lane16 = lax.broadcasted_iota(jnp.int32, (qb, RUN), 1)
    nv, ni = inf16, jnp.zeros((qb, RUN), jnp.int32)
    rvw, riw = rv, ri
    for t in range(TOPK):
        m = jnp.minimum(jnp.min(p0, axis=1), jnp.min(rvw, axis=1))
        mq = m[:, None]
        idb = jnp.min(jnp.where(p0 == mq, q0, BIG), axis=1)
        idr = jnp.min(jnp.where(rvw == mq, riw, BIG), axis=1)
        sel = jnp.minimum(idb, idr)
        selq = sel[:, None]
        from_b = (idb <= idr)[:, None]
        rvw = jnp.where(riw == selq, jnp.inf, rvw)
        shift = from_b & (lane == lax.rem(selq, jnp.int32(128)))
        p0 = jnp.where(shift, p1, p0)
        q0 = jnp.where(shift, q1, q0)
        p1 = jnp.where(shift, p2, p1)
        q1 = jnp.where(shift, q2, q1)
        p2 = jnp.where(shift, p3, p2)
        q2 = jnp.where(shift, q3, q2)
        p3 = jnp.where(shift, jnp.inf, p3)
        nv = jnp.where(lane16 == t, mq, nv)
        ni = jnp.where(lane16 == t, selq, ni)

    runv_ref[...] = nv
    runi_ref[...] = ni

    flag = jnp.any(p3 == jnp.inf)

    @pl.when(flag)
    def _fallback():
        dl, bl = [], []
        for s in range(KB // SUB):
            ds_, b_ = _dist_subtile(q, q_sq, kb, ki, s, k_total)
            dl.append(ds_)
            bl.append(b_)
        nvs, nis = _full_pops(dl, bl, rv, ri, qb)
        runv_ref[...] = nvs
        runi_ref[...] = nis

    @pl.when(ki == kg - 1)
    def _emit():
        nvf = runv_ref[...]
        nif = runi_ref[...]
        d_ref[...] = nvf
        i_ref[...] = nif
        valid = lane16 < TOPK
        mx = jnp.max(jnp.where(valid, nvf, -jnp.inf), axis=1, keepdims=True)
        e = jnp.where(valid, jnp.exp(nvf - mx), 0.0)
        p_ref[...] = e / jnp.sum(e, axis=1, keepdims=True)


def kernel(queries, keys, k):
    del k
    q_n, d = queries.shape
    k_n = keys.shape[0]
    kg = pl.cdiv(k_n, KB)
    kp = kg * KB
    keys_p = jnp.pad(keys, ((0, kp - k_n), (0, 0)))
    qb = q_n if q_n <= 512 else 512
    qg = q_n // qb

    out_shape = [
        jax.ShapeDtypeStruct((q_n, RUN), jnp.float32),
        jax.ShapeDtypeStruct((q_n, RUN), jnp.int32),
        jax.ShapeDtypeStruct((q_n, RUN), jnp.float32),
    ]
    out_specs = [pl.BlockSpec((qb, RUN), lambda qi, ki: (qi, 0))
                 for _ in range(3)]
    dd, ii, pp = pl.pallas_call(
        functools.partial(_body, kg=kg, k_total=k_n, qb=qb),
        grid=(qg, kg),
        in_specs=[
            pl.BlockSpec((qb, d), lambda qi, ki: (qi, 0)),
            pl.BlockSpec((KB, d), lambda qi, ki: (ki, 0)),
        ],
        out_specs=out_specs,
        out_shape=out_shape,
        scratch_shapes=[
            pltpu.VMEM((qb, RUN), jnp.float32),
            pltpu.VMEM((qb, RUN), jnp.int32),
        ],
    )(queries, keys_p)
    return (dd[:, :TOPK], ii[:, :TOPK], pp[:, :TOPK])


# speedup vs baseline: 2.5496x; 2.5496x over previous
import functools

import jax
import jax.numpy as jnp
from jax import lax
from jax.experimental import pallas as pl
from jax.experimental.pallas import tpu as pltpu

KB = 4096
SUB = 1024
TOPK = 10
RUN = 16
BIG = 2**31 - 1


def _dist_subtile(q, q_sq, kb, ki, s, k_total):
    kb_s = kb[s * SUB:(s + 1) * SUB, :]
    dots = lax.dot_general(q, kb_s, (((1,), (1,)), ((), ())),
                           preferred_element_type=jnp.float32)
    ones = jnp.ones((1, 128), jnp.float32)
    k_sq = lax.dot_general(ones, kb_s * kb_s, (((1,), (1,)), ((), ())),
                           preferred_element_type=jnp.float32,
                           precision=lax.Precision.HIGHEST)
    base = ki * KB + s * SUB
    col1 = base + lax.broadcasted_iota(jnp.int32, (1, SUB), 1)
    k_sq = jnp.where(col1 < k_total, k_sq, jnp.inf)
    return (q_sq + k_sq) - 2.0 * dots, base


def _full_pops(dists, bases, rv, ri, qb):
    nv = jnp.full((qb, RUN), jnp.inf, jnp.float32)
    ni = jnp.zeros((qb, RUN), jnp.int32)
    lane16 = lax.broadcasted_iota(jnp.int32, (qb, RUN), 1)
    dists = list(dists)
    for t in range(TOPK):
        m = jnp.min(rv, axis=1)
        for d in dists:
            m = jnp.minimum(m, jnp.min(d, axis=1))
        mq = m[:, None]
        sel = jnp.min(jnp.where(rv == mq, ri, BIG), axis=1)
        for j, d in enumerate(dists):
            colj = bases[j] + lax.broadcasted_iota(jnp.int32, (qb, SUB), 1)
            sel = jnp.minimum(
                sel, jnp.min(jnp.where(d == mq, colj, BIG), axis=1))
        selq = sel[:, None]
        rv = jnp.where(ri == selq, jnp.inf, rv)
        for j, d in enumerate(dists):
            colj = bases[j] + lax.broadcasted_iota(jnp.int32, (qb, SUB), 1)
            dists[j] = jnp.where(colj == selq, jnp.inf, d)
        nv = jnp.where(lane16 == t, mq, nv)
        ni = jnp.where(lane16 == t, selq, ni)
    return nv, ni


def _body(q_ref, kb_ref, d_ref, i_ref, p_ref, runv_ref, runi_ref, *,
          kg, k_total, qb):
    ki = pl.program_id(1)

    q = q_ref[...]
    kb = kb_ref[...]
    q_sq = jnp.sum(q * q, axis=1, keepdims=True)

    inf16 = jnp.full((qb, RUN), jnp.inf, jnp.float32)
    rv = jnp.where(ki == 0, inf16, runv_ref[...])
    ri = jnp.where(ki == 0, jnp.zeros((qb, RUN), jnp.int32), runi_ref[...])

    lane = lax.broadcasted_iota(jnp.int32, (qb, 128), 1)
    infs = jnp.full((qb, 128), jnp.inf, jnp.float32)
    zi = jnp.zeros((qb, 128), jnp.int32)
    p0, p1, p2, p3 = infs, infs, infs, infs
    q0, q1, q2, q3 = zi, zi, zi, zi

    for s in range(KB // SUB):
        dist_s, base = _dist_subtile(q, q_sq, kb, ki, s, k_total)
        for g in range(SUB // 128):
            d = dist_s[:, g * 128:(g + 1) * 128]
            cid = (base + g * 128) + lane
            c1 = d < p0
            t1 = jnp.maximum(d, p0)
            ti1 = jnp.where(c1, q0, cid)
            p0 = jnp.minimum(d, p0)
            q0 = jnp.where(c1, cid, q0)
            c2 = t1 < p1
            t2 = jnp.maximum(t1, p1)
            ti2 = jnp.where(c2, q1, ti1)
            p1 = jnp.minimum(t1, p1)
            q1 = jnp.where(c2, ti1, q1)
            c3 = t2 < p2
            t3 = jnp.maximum(t2, p2)
            ti3 = jnp.where(c3, q2, ti2)
            p2 = jnp.minimum(t2, p2)
            q2 = jnp.where(c3, ti2, q2)
            c4 = t3 < p3
            p3 = jnp.minimum(t3, p3)
            q3 = jnp.where(c4, ti3, q3)

    lane16 = lax.broadcasted_iota(jnp.int32, (qb, RUN), 1)
    nv, ni = inf16, jnp.zeros((qb, RUN), jnp.int32)
    rvw, riw = rv, ri
    for t in range(TOPK):
        m = jnp.minimum(jnp.min(p0, axis=1), jnp.min(rvw, axis=1))
        mq = m[:, None]
        idb = jnp.min(jnp.where(p0 == mq, q0, BIG), axis=1)
        idr = jnp.min(jnp.where(rvw == mq, riw, BIG), axis=1)
        sel = jnp.minimum(idb, idr)
        selq = sel[:, None]
        from_b = (idb <= idr)[:, None]
        rvw = jnp.where(riw == selq, jnp.inf, rvw)
        shift = from_b & (lane == lax.rem(selq, jnp.int32(128)))
        p0 = jnp.where(shift, p1, p0)
        q0 = jnp.where(shift, q1, q0)
        p1 = jnp.where(shift, p2, p1)
        q1 = jnp.where(shift, q2, q1)
        p2 = jnp.where(shift, p3, p2)
        q2 = jnp.where(shift, q3, q2)
        p3 = jnp.where(shift, jnp.inf, p3)
        nv = jnp.where(lane16 == t, mq, nv)
        ni = jnp.where(lane16 == t, selq, ni)

    runv_ref[...] = nv
    runi_ref[...] = ni

    flag = jnp.any(p0 == jnp.inf)

    @pl.when(flag)
    def _fallback():
        dl, bl = [], []
        for s in range(KB // SUB):
            ds_, b_ = _dist_subtile(q, q_sq, kb, ki, s, k_total)
            dl.append(ds_)
            bl.append(b_)
        nvs, nis = _full_pops(dl, bl, rv, ri, qb)
        runv_ref[...] = nvs
        runi_ref[...] = nis

    @pl.when(ki == kg - 1)
    def _emit():
        nvf = runv_ref[...]
        nif = runi_ref[...]
        d_ref[...] = nvf
        i_ref[...] = nif
        valid = lane16 < TOPK
        mx = jnp.max(jnp.where(valid, nvf, -jnp.inf), axis=1, keepdims=True)
        e = jnp.where(valid, jnp.exp(nvf - mx), 0.0)
        p_ref[...] = e / jnp.sum(e, axis=1, keepdims=True)


def kernel(queries, keys, k):
    del k
    q_n, d = queries.shape
    k_n = keys.shape[0]
    kg = pl.cdiv(k_n, KB)
    kp = kg * KB
    keys_p = jnp.pad(keys, ((0, kp - k_n), (0, 0)))
    qb = q_n if q_n <= 512 else 512
    qg = q_n // qb

    out_shape = [
        jax.ShapeDtypeStruct((q_n, RUN), jnp.float32),
        jax.ShapeDtypeStruct((q_n, RUN), jnp.int32),
        jax.ShapeDtypeStruct((q_n, RUN), jnp.float32),
    ]
    out_specs = [pl.BlockSpec((qb, RUN), lambda qi, ki: (qi, 0))
                 for _ in range(3)]
    dd, ii, pp = pl.pallas_call(
        functools.partial(_body, kg=kg, k_total=k_n, qb=qb),
        grid=(qg, kg),
        in_specs=[
            pl.BlockSpec((qb, d), lambda qi, ki: (qi, 0)),
            pl.BlockSpec((KB, d), lambda qi, ki: (ki, 0)),
        ],
        out_specs=out_specs,
        out_shape=out_shape,
        scratch_shapes=[
            pltpu.VMEM((qb, RUN), jnp.float32),
            pltpu.VMEM((qb, RUN), jnp.int32),
        ],
    )(queries, keys_p)
    return (dd[:, :TOPK], ii[:, :TOPK], pp[:, :TOPK])


# speedup vs baseline: 2.5850x; 1.0139x over previous
import functools

import jax
import jax.numpy as jnp
from jax import lax
from jax.experimental import pallas as pl
from jax.experimental.pallas import tpu as pltpu

KB = 4096
SUB = 1024
TOPK = 10
RUN = 16
BIG = 2**31 - 1


def _dist_subtile(q, q_sq, kb, ki, s, k_total):
    kb_s = kb[s * SUB:(s + 1) * SUB, :]
    dots = lax.dot_general(q, kb_s, (((1,), (1,)), ((), ())),
                           preferred_element_type=jnp.float32)
    ones = jnp.ones((1, 128), jnp.float32)
    k_sq = lax.dot_general(ones, kb_s * kb_s, (((1,), (1,)), ((), ())),
                           preferred_element_type=jnp.float32,
                           precision=lax.Precision.HIGHEST)
    base = ki * KB + s * SUB
    col1 = base + lax.broadcasted_iota(jnp.int32, (1, SUB), 1)
    k_sq = jnp.where(col1 < k_total, k_sq, jnp.inf)
    return (q_sq + k_sq) - 2.0 * dots, base


def _full_pops(dists, bases, rv, ri, qb):
    nv = jnp.full((qb, RUN), jnp.inf, jnp.float32)
    ni = jnp.zeros((qb, RUN), jnp.int32)
    lane16 = lax.broadcasted_iota(jnp.int32, (qb, RUN), 1)
    dists = list(dists)
    for t in range(TOPK):
        m = jnp.min(rv, axis=1)
        for d in dists:
            m = jnp.minimum(m, jnp.min(d, axis=1))
        mq = m[:, None]
        sel = jnp.min(jnp.where(rv == mq, ri, BIG), axis=1)
        for j, d in enumerate(dists):
            colj = bases[j] + lax.broadcasted_iota(jnp.int32, (qb, SUB), 1)
            sel = jnp.minimum(
                sel, jnp.min(jnp.where(d == mq, colj, BIG), axis=1))
        selq = sel[:, None]
        rv = jnp.where(ri == selq, jnp.inf, rv)
        for j, d in enumerate(dists):
            colj = bases[j] + lax.broadcasted_iota(jnp.int32, (qb, SUB), 1)
            dists[j] = jnp.where(colj == selq, jnp.inf, d)
        nv = jnp.where(lane16 == t, mq, nv)
        ni = jnp.where(lane16 == t, selq, ni)
    return nv, ni


def _body(q_ref, kb_ref, d_ref, i_ref, p_ref, runv_ref, runi_ref, *,
          kg, k_total, qb):
    ki = pl.program_id(1)

    q = q_ref[...]
    kb = kb_ref[...]
    q_sq = jnp.sum(q * q, axis=1, keepdims=True)

    inf16 = jnp.full((qb, RUN), jnp.inf, jnp.float32)
    rv = jnp.where(ki == 0, inf16, runv_ref[...])
    ri = jnp.where(ki == 0, jnp.zeros((qb, RUN), jnp.int32), runi_ref[...])

    lane = lax.broadcasted_iota(jnp.int32, (qb, 128), 1)
    infs = jnp.full((qb, 128), jnp.inf, jnp.float32)
    zi = jnp.zeros((qb, 128), jnp.int32)
    p0, p1, p2, p3 = infs, infs, infs, infs
    q0, q1, q2, q3 = zi, zi, zi, zi

    for s in range(KB // SUB):
        dist_s, base = _dist_subtile(q, q_sq, kb, ki, s, k_total)
        for g in range(SUB // 128):
            d = dist_s[:, g * 128:(g + 1) * 128]
            cid = (base + g * 128) + lane
            c1 = d < p0
            t1 = jnp.maximum(d, p0)
            ti1 = jnp.where(c1, q0, cid)
            p0 = jnp.minimum(d, p0)
            q0 = jnp.where(c1, cid, q0)
            c2 = t1 < p1
            t2 = jnp.maximum(t1, p1)
            ti2 = jnp.where(c2, q1, ti1)
            p1 = jnp.minimum(t1, p1)
            q1 = jnp.where(c2, ti1, q1)
            c3 = t2 < p2
            t3 = jnp.maximum(t2, p2)
            ti3 = jnp.where(c3, q2, ti2)
            p2 = jnp.minimum(t2, p2)
            q2 = jnp.where(c3, ti2, q2)
            c4 = t3 < p3
            p3 = jnp.minimum(t3, p3)
            q3 = jnp.where(c4, ti3, q3)

    lane16 = lax.broadcasted_iota(jnp.int32, (qb, RUN), 1)
    nv, ni = inf16, jnp.zeros((qb, RUN), jnp.int32)
    rvw, riw = rv, ri
    for t in range(TOPK):
        m = jnp.minimum(jnp.min(p0, axis=1), jnp.min(rvw, axis=1))
        mq = m[:, None]
        idb = jnp.min(jnp.where(p0 == mq, q0, BIG), axis=1)
        idr = jnp.min(jnp.where(rvw == mq, riw, BIG), axis=1)
        sel = jnp.minimum(idb, idr)
        selq = sel[:, None]
        from_b = (idb <= idr)[:, None]
        rvw = jnp.where(riw == selq, jnp.inf, rvw)
        shift = from_b & (q0 == selq)
        p0 = jnp.where(shift, p1, p0)
        q0 = jnp.where(shift, q1, q0)
        p1 = jnp.where(shift, p2, p1)
        q1 = jnp.where(shift, q2, q1)
        p2 = jnp.where(shift, p3, p2)
        q2 = jnp.where(shift, q3, q2)
        p3 = jnp.where(shift, jnp.inf, p3)
        nv = jnp.where(lane16 == t, mq, nv)
        ni = jnp.where(lane16 == t, selq, ni)

    runv_ref[...] = nv
    runi_ref[...] = ni

    flag = jnp.any(p0 == jnp.inf)

    @pl.when(flag)
    def _fallback():
        dl, bl = [], []
        for s in range(KB // SUB):
            ds_, b_ = _dist_subtile(q, q_sq, kb, ki, s, k_total)
            dl.append(ds_)
            bl.append(b_)
        nvs, nis = _full_pops(dl, bl, rv, ri, qb)
        runv_ref[...] = nvs
        runi_ref[...] = nis

    @pl.when(ki == kg - 1)
    def _emit():
        nvf = runv_ref[...]
        nif = runi_ref[...]
        d_ref[...] = nvf
        i_ref[...] = nif
        valid = lane16 < TOPK
        mx = jnp.max(jnp.where(valid, nvf, -jnp.inf), axis=1, keepdims=True)
        e = jnp.where(valid, jnp.exp(nvf - mx), 0.0)
        p_ref[...] = e / jnp.sum(e, axis=1, keepdims=True)


def kernel(queries, keys, k):
    del k
    q_n, d = queries.shape
    k_n = keys.shape[0]
    kg = pl.cdiv(k_n, KB)
    kp = kg * KB
    keys_p = jnp.pad(keys, ((0, kp - k_n), (0, 0)))
    qb = q_n if q_n <= 512 else 512
    qg = q_n // qb

    out_shape = [
        jax.ShapeDtypeStruct((q_n, RUN), jnp.float32),
        jax.ShapeDtypeStruct((q_n, RUN), jnp.int32),
        jax.ShapeDtypeStruct((q_n, RUN), jnp.float32),
    ]
    out_specs = [pl.BlockSpec((qb, RUN), lambda qi, ki: (qi, 0))
                 for _ in range(3)]
    dd, ii, pp = pl.pallas_call(
        functools.partial(_body, kg=kg, k_total=k_n, qb=qb),
        grid=(qg, kg),
        in_specs=[
            pl.BlockSpec((qb, d), lambda qi, ki: (qi, 0)),
            pl.BlockSpec((KB, d), lambda qi, ki: (ki, 0)),
        ],
        out_specs=out_specs,
        out_shape=out_shape,
        scratch_shapes=[
            pltpu.VMEM((qb, RUN), jnp.float32),
            pltpu.VMEM((qb, RUN), jnp.int32),
        ],
    )(queries, keys_p)
    return (dd[:, :TOPK], ii[:, :TOPK], pp[:, :TOPK])
